# Initial kernel scaffold; baseline (speedup 1.0000x reference)
import jax
import jax.numpy as jnp
from jax.experimental import pallas as pl


def kernel(a, b):
    raise NotImplementedError("write your pallas kernel here")


# speedup vs baseline: 1.2051x; 1.2051x over previous
import jax
import jax.numpy as jnp
from jax.experimental import pallas as pl
from jax.experimental.pallas import tpu as pltpu

_TM = 1024
_TN = 1024


def _mm_kernel(a_ref, b_ref, o_ref):
    o_ref[...] = jnp.dot(
        a_ref[...], b_ref[...], preferred_element_type=jnp.float32
    )


def _ceil_to(x, m):
    return ((x + m - 1) // m) * m


def kernel(a, b):
    M, K = a.shape
    K2, N = b.shape
    assert K == K2

    a16 = a.astype(jnp.bfloat16)
    b16 = b.astype(jnp.bfloat16)

    Mp, Kp, Np = _ceil_to(M, 8), _ceil_to(K, 128), _ceil_to(N, 128)
    if (Mp, Kp) != (M, K):
        a16 = jnp.pad(a16, ((0, Mp - M), (0, Kp - K)))
    if (Kp, Np) != (K, N):
        b16 = jnp.pad(b16, ((0, Kp - K), (0, Np - N)))

    tm = min(_TM, Mp)
    tn = min(_TN, Np)
    grid_m = -(-Mp // tm)
    grid_n = -(-Np // tn)
    Mp2, Np2 = grid_m * tm, grid_n * tn
    if (Mp2, Np2) != (Mp, Np):
        a16 = jnp.pad(a16, ((0, Mp2 - Mp), (0, 0)))
        b16 = jnp.pad(b16, ((0, 0), (0, Np2 - Np)))

    out = pl.pallas_call(
        _mm_kernel,
        out_shape=jax.ShapeDtypeStruct((Mp2, Np2), jnp.float32),
        grid=(grid_m, grid_n),
        in_specs=[
            pl.BlockSpec((tm, Kp), lambda i, j: (i, 0)),
            pl.BlockSpec((Kp, tn), lambda i, j: (0, j)),
        ],
        out_specs=pl.BlockSpec((tm, tn), lambda i, j: (i, j)),
        compiler_params=pltpu.CompilerParams(
            dimension_semantics=("parallel", "parallel"),
            vmem_limit_bytes=56 * 1024 * 1024,
        ),
        cost_estimate=pl.CostEstimate(
            flops=2 * M * N * K,
            transcendentals=0,
            bytes_accessed=(M * K + K * N) * 2 + M * N * 4,
        ),
    )(a16, b16)

    if (Mp2, Np2) != (M, N):
        out = out[:M, :N]
    return out


# speedup vs baseline: 1.3184x; 1.0940x over previous
import jax
import jax.numpy as jnp
from jax.experimental import pallas as pl
from jax.experimental.pallas import tpu as pltpu

_TM = 1024
_TN = 512


def _mm_kernel(a_ref, b_ref, o_ref):
    o_ref[...] = jnp.dot(
        a_ref[...],
        b_ref[...].astype(jnp.bfloat16),
        preferred_element_type=jnp.float32,
    )


def _ceil_to(x, m):
    return ((x + m - 1) // m) * m


def kernel(a, b):
    M, K = a.shape
    K2, N = b.shape
    assert K == K2

    a16 = a.astype(jnp.bfloat16)
    b16 = b

    Mp, Kp, Np = _ceil_to(M, 8), _ceil_to(K, 128), _ceil_to(N, 128)
    if (Mp, Kp) != (M, K):
        a16 = jnp.pad(a16, ((0, Mp - M), (0, Kp - K)))
    if (Kp, Np) != (K, N):
        b16 = jnp.pad(b16, ((0, Kp - K), (0, Np - N)))

    tm = min(_TM, Mp)
    tn = min(_TN, Np)
    grid_m = -(-Mp // tm)
    grid_n = -(-Np // tn)
    Mp2, Np2 = grid_m * tm, grid_n * tn
    if (Mp2, Np2) != (Mp, Np):
        a16 = jnp.pad(a16, ((0, Mp2 - Mp), (0, 0)))
        b16 = jnp.pad(b16, ((0, 0), (0, Np2 - Np)))

    out = pl.pallas_call(
        _mm_kernel,
        out_shape=jax.ShapeDtypeStruct((Mp2, Np2), jnp.float32),
        grid=(grid_m, grid_n),
        in_specs=[
            pl.BlockSpec((tm, Kp), lambda i, j: (i, 0)),
            pl.BlockSpec((Kp, tn), lambda i, j: (0, j)),
        ],
        out_specs=pl.BlockSpec((tm, tn), lambda i, j: (i, j)),
        compiler_params=pltpu.CompilerParams(
            dimension_semantics=("parallel", "parallel"),
            vmem_limit_bytes=60 * 1024 * 1024,
        ),
        cost_estimate=pl.CostEstimate(
            flops=2 * M * N * K,
            transcendentals=0,
            bytes_accessed=M * K * 2 + K * N * 4 + M * N * 4,
        ),
    )(a16, b16)

    if (Mp2, Np2) != (M, N):
        out = out[:M, :N]
    return out


# speedup vs baseline: 1.4850x; 1.1263x over previous
import jax
import jax.numpy as jnp
from jax.experimental import pallas as pl
from jax.experimental.pallas import tpu as pltpu

_TM = 1024
_TN = 512


def _mm_kernel(a_ref, b_ref, o_ref):
    o_ref[...] = jnp.dot(
        a_ref[...].astype(jnp.bfloat16),
        b_ref[...].astype(jnp.bfloat16),
        preferred_element_type=jnp.float32,
    )


def _ceil_to(x, m):
    return ((x + m - 1) // m) * m


def kernel(a, b):
    M, K = a.shape
    K2, N = b.shape
    assert K == K2

    a16 = a
    b16 = b

    Mp, Kp, Np = _ceil_to(M, 8), _ceil_to(K, 128), _ceil_to(N, 128)
    if (Mp, Kp) != (M, K):
        a16 = jnp.pad(a16, ((0, Mp - M), (0, Kp - K)))
    if (Kp, Np) != (K, N):
        b16 = jnp.pad(b16, ((0, Kp - K), (0, Np - N)))

    tm = min(_TM, Mp)
    tn = min(_TN, Np)
    grid_m = -(-Mp // tm)
    grid_n = -(-Np // tn)
    Mp2, Np2 = grid_m * tm, grid_n * tn
    if (Mp2, Np2) != (Mp, Np):
        a16 = jnp.pad(a16, ((0, Mp2 - Mp), (0, 0)))
        b16 = jnp.pad(b16, ((0, 0), (0, Np2 - Np)))

    out = pl.pallas_call(
        _mm_kernel,
        out_shape=jax.ShapeDtypeStruct((Mp2, Np2), jnp.float32),
        grid=(grid_m, grid_n),
        in_specs=[
            pl.BlockSpec((tm, Kp), lambda i, j: (i, 0)),
            pl.BlockSpec((Kp, tn), lambda i, j: (0, j)),
        ],
        out_specs=pl.BlockSpec((tm, tn), lambda i, j: (i, j)),
        compiler_params=pltpu.CompilerParams(
            dimension_semantics=("parallel", "parallel"),
            vmem_limit_bytes=60 * 1024 * 1024,
        ),
        cost_estimate=pl.CostEstimate(
            flops=2 * M * N * K,
            transcendentals=0,
            bytes_accessed=M * K * 2 + K * N * 4 + M * N * 4,
        ),
    )(a16, b16)

    if (Mp2, Np2) != (M, N):
        out = out[:M, :N]
    return out
